# baseline (speedup 1.0000x reference)
import functools
import math

import jax
import jax.numpy as jnp
from jax import lax
from jax.experimental import pallas as pl
from jax.experimental.pallas import tpu as pltpu
from jax.experimental.pallas import tpu_sc as plsc

_HIDDEN = 768
_PATCH_NUM = 84
_SELECT_NUM = 42
_B = 4
_C = 12
_S = 1024
_H = 32
_PAD_SEL = 64

_HIGH = lax.Precision.HIGHEST


def _shift_lanes(a, s):
    z = jnp.zeros((a.shape[0], abs(s)), a.dtype)
    if s > 0:
        return jnp.concatenate([a[:, s:], z], axis=1)
    if s < 0:
        return jnp.concatenate([z, a[:, :s]], axis=1)
    return a


def _cumsum_lanes_excl(a):
    acc = a
    d = 1
    while d < a.shape[1]:
        acc = acc + _shift_lanes(acc, -d)
        d *= 2
    return acc - a


def _compute_body(sn_ref, score_ref, row0_ref, w1_ref, w2_ref,
                  row0_out, patch_out, gidx_out):
    score = score_ref[...]

    u = lax.bitcast_convert_type(score, jnp.uint32)
    neg = (u >> jnp.uint32(31)) > jnp.uint32(0)
    ukey = jnp.where(neg, ~u, u | jnp.uint32(0x80000000))

    def bs_body(i, m):
        cand = m | (jnp.uint32(0x80000000) >> i.astype(jnp.uint32))
        cnt = jnp.sum((ukey >= cand).astype(jnp.int32), axis=1, keepdims=True)
        return jnp.where(cnt >= _PATCH_NUM, cand, m)

    thr_key = lax.fori_loop(0, 32, bs_body, jnp.zeros((48, 1), jnp.uint32))

    gt = ukey > thr_key
    eq = ukey == thr_key
    cnt_gt = jnp.sum(gt.astype(jnp.int32), axis=1, keepdims=True)
    need = _PATCH_NUM - cnt_gt

    eq_rank = _cumsum_lanes_excl(eq.astype(jnp.int32))
    mask = gt | (eq & (eq_rank < need))
    mask_f = mask.astype(jnp.float32)
    new_score = jnp.where(mask, score, score * 0.7)

    g_r = lax.broadcasted_iota(jnp.int32, (4, 48), 0)
    g_c = lax.broadcasted_iota(jnp.int32, (4, 48), 1)
    grp = jnp.where(g_c // _C == g_r, 1.0, 0.0).astype(jnp.float32)
    s1 = lax.dot_general(grp, new_score, (((1,), (0,)), ((), ())),
                         precision=_HIGH)
    count = lax.dot_general(grp, mask_f, (((1,), (0,)), ((), ())),
                            precision=_HIGH)
    pw = s1 * (1.0 / _C)

    thr = jnp.mean(s1, axis=1, keepdims=True)
    binary = (s1 > thr).astype(jnp.float32)
    m_arr = pw * binary

    lane = lax.broadcasted_iota(jnp.int32, (4, 1024), 1)
    mx = jnp.max(m_arr, axis=1, keepdims=True)
    idx_max = jnp.min(jnp.where(m_arr == mx, lane, 1024), axis=1,
                      keepdims=True)

    ai = (idx_max // _H).astype(jnp.float32)
    aj = (idx_max % _H).astype(jnp.float32)
    pi = (lane // _H).astype(jnp.float32)
    pj = (lane % _H).astype(jnp.float32)
    ri = (pi - ai) * (1.0 / _H)
    rj = (pj - aj) * (1.0 / _H)
    dist = jnp.sqrt(ri * ri + rj * rj)
    ang = (jnp.arctan2(rj, ri) * (1.0 / math.pi) + 1.0) * 0.5

    cw = jnp.sum(pw * dist, axis=1, keepdims=True)
    ca = jnp.sum(pw * ang, axis=1, keepdims=True)
    pw2 = pw * pw
    p_pos = jnp.sum(jnp.where(pw > 0, pw2, 0.0), axis=1, keepdims=True)
    p_neg = jnp.sum(jnp.where(pw < 0, pw2, 0.0), axis=1, keepdims=True)

    w1 = w1_ref[...]
    t = cw * w1[0:1, :] + ca * w1[1:2, :]
    v = t * jnp.where(t > 0, p_pos, p_neg)
    w = lax.dot_general(v, w2_ref[...], (((1,), (0,)), ((), ())),
                        precision=_HIGH)
    pw_anchor = jnp.sum(jnp.where(lane == idx_max, pw, 0.0), axis=1,
                        keepdims=True)
    z = pw_anchor * w
    sinfo = jnp.where(z >= 0, z, 0.2 * z)
    row0_out[...] = row0_ref[...] + sinfo

    jmod = lane & 31
    m_jm1 = (jmod > 0).astype(jnp.float32)
    m_jp1 = (jmod < 31).astype(jnp.float32)
    csm = jnp.zeros((_B, _S), jnp.float32)
    for di in (-1, 0, 1):
        for dj in (-1, 0, 1):
            wgt = float((2 - abs(di)) * (2 - abs(dj)))
            term = _shift_lanes(count, 32 * di + dj) * wgt
            if dj == -1:
                term = term * m_jm1
            elif dj == 1:
                term = term * m_jp1
            csm = csm + term
    ci = csm.astype(jnp.int32)
    key2 = ci * 1024 + (1023 - lane)

    keep = jnp.minimum(jnp.int32(_SELECT_NUM), sn_ref[0, 0])
    r_lane = lax.broadcasted_iota(jnp.int32, (_B, _PAD_SEL), 1)
    patch_i = jnp.zeros((_B, _PAD_SEL), jnp.int32)
    for r in range(_SELECT_NUM):
        mxk = jnp.max(key2, axis=1, keepdims=True)
        am = jnp.min(jnp.where(key2 == mxk, lane, 2048), axis=1,
                     keepdims=True)
        patch_i = jnp.where(r_lane == r, am + 1, patch_i)
        key2 = jnp.where(lane == am, -1, key2)
    patch_i = jnp.where(r_lane < keep, patch_i, 0)
    patch_out[...] = patch_i
    b_iota = lax.broadcasted_iota(jnp.int32, (_B, _PAD_SEL), 0)
    gidx_out[...] = patch_i + b_iota * (_S + 1)


def _copy_body(hid_ref, row0_ref, out_ref):
    out_ref[...] = hid_ref[...]
    out_ref[0, 0, :] = row0_ref[0, 0, :]


def _gather_body(tab_ref, idx_ref, out_ref, idx_v, rows_v, sem):
    nc = 2
    wid = lax.axis_index("s") * nc + lax.axis_index("c")
    per = (_B * _PAD_SEL) // (nc * 16)
    base = wid * per
    pltpu.sync_copy(idx_ref.at[pl.ds(base, per)], idx_v)
    pltpu.async_copy(tab_ref.at[idx_v], rows_v, sem).wait()
    pltpu.sync_copy(rows_v, out_ref.at[pl.ds(base, per)])


def kernel(hidden_states, x, contribution, select_num, W1, W2):
    del contribution
    score = x[:, :, 0, 1:].reshape(_B * _C, _S)
    row0 = hidden_states[:, 0, :]
    sn = jnp.asarray(select_num, jnp.int32).reshape(1, 1)

    row0_new, patch_pad, gidx = pl.pallas_call(
        _compute_body,
        in_specs=[pl.BlockSpec(memory_space=pltpu.SMEM)] +
                 [pl.BlockSpec(memory_space=pltpu.VMEM)] * 4,
        out_specs=[pl.BlockSpec(memory_space=pltpu.VMEM)] * 3,
        out_shape=[
            jax.ShapeDtypeStruct((_B, _HIDDEN), jnp.float32),
            jax.ShapeDtypeStruct((_B, _PAD_SEL), jnp.int32),
            jax.ShapeDtypeStruct((_B, _PAD_SEL), jnp.int32),
        ],
    )(sn, score, row0, W1, W2)

    hidden_out = pl.pallas_call(
        _copy_body,
        grid=(_B,),
        in_specs=[
            pl.BlockSpec((1, _S + 1, _HIDDEN), lambda b: (b, 0, 0)),
            pl.BlockSpec((1, 1, _HIDDEN), lambda b: (b, 0, 0)),
        ],
        out_specs=pl.BlockSpec((1, _S + 1, _HIDDEN), lambda b: (b, 0, 0)),
        out_shape=jax.ShapeDtypeStruct((_B, _S + 1, _HIDDEN), jnp.float32),
    )(hidden_states, row0_new.reshape(_B, 1, _HIDDEN))

    mesh = plsc.VectorSubcoreMesh(core_axis_name="c", subcore_axis_name="s")
    gather = functools.partial(
        pl.kernel,
        mesh=mesh,
        out_type=jax.ShapeDtypeStruct((_B * _PAD_SEL, _HIDDEN), jnp.float32),
        scratch_types=[
            pltpu.VMEM(((_B * _PAD_SEL) // 32,), jnp.int32),
            pltpu.VMEM(((_B * _PAD_SEL) // 32, _HIDDEN), jnp.float32),
            pltpu.SemaphoreType.DMA,
        ],
    )(_gather_body)
    flat = gather(hidden_states.reshape(_B * (_S + 1), _HIDDEN),
                  gidx.reshape(_B * _PAD_SEL))
    sel = flat.reshape(_B, _PAD_SEL, _HIDDEN)[:, :_SELECT_NUM, :]

    patch_idx = patch_pad[:, :_SELECT_NUM]
    selected = jnp.where((patch_idx == 0)[:, :, None],
                         hidden_out[:, 0, :][:, None, :], sel)
    return hidden_out, selected, patch_idx
